# baseline (speedup 1.0000x reference)
import functools

import jax
import jax.numpy as jnp
from jax import lax
from jax.experimental import pallas as pl
from jax.experimental.pallas import tpu as pltpu
from jax.experimental.pallas import tpu_sc as plsc

N_PAD = 2048
TP = 6


def _rne_bf16_bits(bits):
    bits = bits.astype(jnp.uint32)
    return (bits + 0x7FFF + ((bits >> 16) & 1)) >> 16


def _table_body(c_ref, o_ref):
    c = c_ref[...]
    rho0, rho1, rho2 = c[0:1], c[1:2], c[2:3]
    x, y, z = c[3:4], c[4:5], c[5:6]
    th2 = x * x + y * y + z * z
    small = th2 < 1e-8
    th2s = jnp.where(small, 1.0, th2)
    th = jnp.sqrt(th2s)
    s = jnp.sin(th)
    co = jnp.cos(th)
    A = jnp.where(small, 1.0 - th2 / 6.0, s / th)
    B = jnp.where(small, 0.5 - th2 / 24.0, (1.0 - co) / th2s)
    Cc = jnp.where(small, 1.0 / 6.0 - th2 / 120.0, (th - s) / (th2s * th))
    xx, yy, zz = x * x, y * y, z * z
    xy, xz, yz = x * y, x * z, y * z
    R00 = 1.0 - B * (yy + zz)
    R01 = -A * z + B * xy
    R02 = A * y + B * xz
    R10 = A * z + B * xy
    R11 = 1.0 - B * (xx + zz)
    R12 = -A * x + B * yz
    R20 = -A * y + B * xz
    R21 = A * x + B * yz
    R22 = 1.0 - B * (xx + yy)
    J00 = 1.0 - Cc * (yy + zz)
    J01 = -B * z + Cc * xy
    J02 = B * y + Cc * xz
    J10 = B * z + Cc * xy
    J11 = 1.0 - Cc * (xx + zz)
    J12 = -B * x + Cc * yz
    J20 = -B * y + Cc * xz
    J21 = B * x + Cc * yz
    J22 = 1.0 - Cc * (xx + yy)
    t0 = J00 * rho0 + J01 * rho1 + J02 * rho2
    t1 = J10 * rho0 + J11 * rho1 + J12 * rho2
    t2 = J20 * rho0 + J21 * rho1 + J22 * rho2
    rows = (t0, t1, t2, R00, R01, R02, R10, R11, R12, R20, R21, R22)
    for p in range(TP):
        lo = _rne_bf16_bits(lax.bitcast_convert_type(rows[2 * p], jnp.uint32))
        hi = _rne_bf16_bits(lax.bitcast_convert_type(rows[2 * p + 1], jnp.uint32))
        o_ref[p:p + 1, :] = lax.bitcast_convert_type(lo | (hi << 16), jnp.int32)


_table_call = pl.pallas_call(
    _table_body,
    out_shape=jax.ShapeDtypeStruct((TP, N_PAD), jnp.int32),
)


@functools.lru_cache(maxsize=None)
def _make_sc(n_tiles, n_frames, n_chunks, unroll, n_buf=4):
    info = plsc.get_sparse_core_info()
    nc, ns = info.num_cores, info.num_subcores
    nw = nc * ns
    tiles_w = n_tiles // nw
    assert tiles_w * nw == n_tiles and tiles_w % n_chunks == 0
    tiles_per_chunk = tiles_w // n_chunks
    cw = tiles_per_chunk * 1024
    ciw = tiles_per_chunk * 128
    n_rays_w = tiles_w * 128
    mesh = plsc.VectorSubcoreMesh(core_axis_name="c", subcore_axis_name="s")
    interleaved = plsc.PackFormat.INTERLEAVED

    @functools.partial(
        pl.kernel,
        out_type=jax.ShapeDtypeStruct((n_tiles * 1024,), jnp.float32),
        mesh=mesh,
        compiler_params=pltpu.CompilerParams(needs_layout_passes=False),
        scratch_types=[
            pltpu.VMEM((N_PAD * TP,), jnp.int32),
            [pltpu.VMEM((ciw + 16,), jnp.int32) for _ in range(n_buf)],
            [pltpu.VMEM((ciw + 16,), jnp.int32) for _ in range(n_buf)],
            [pltpu.VMEM((cw,), jnp.float32) for _ in range(n_buf)],
            [pltpu.VMEM((cw,), jnp.float32) for _ in range(n_buf)],
            [pltpu.SemaphoreType.DMA for _ in range(n_buf)],
            [pltpu.SemaphoreType.DMA for _ in range(n_buf)],
        ],
    )
    def sc_k(table_hbm, rays_hbm, idx_hbm, msk_hbm, out_hbm,
             tab_v, idx_vs, msk_vs, ray_vs, out_vs, in_sems, out_sems):
        wid = lax.axis_index("s") * nc + lax.axis_index("c")
        iota = lax.iota(jnp.int32, 16)
        zero16 = jnp.zeros((16,), jnp.float32)

        def fire_in(ci):
            b = ci % n_buf
            rbase = wid * n_rays_w + ci * ciw
            wbase = wid * tiles_w * 1024 + ci * cw
            return (
                pltpu.async_copy(idx_hbm.at[pl.ds(rbase, ciw)],
                                 idx_vs[b].at[pl.ds(0, ciw)], in_sems[b]),
                pltpu.async_copy(msk_hbm.at[pl.ds(rbase, ciw)],
                                 msk_vs[b].at[pl.ds(0, ciw)], in_sems[b]),
                pltpu.async_copy(rays_hbm.at[pl.ds(wbase, cw)], ray_vs[b], in_sems[b]),
            )

        in_handles = {ci: fire_in(ci) for ci in range(min(n_buf, n_chunks))}
        pltpu.sync_copy(table_hbm, tab_v)

        out_handles = {}
        for ci in range(n_chunks):
            b = ci % n_buf
            idx_v, msk_v = idx_vs[b], msk_vs[b]
            ray_v, out_v = ray_vs[b], out_vs[b]
            for h in in_handles.pop(ci):
                h.wait()
            if ci - n_buf >= 0:
                out_handles.pop(ci - n_buf).wait()

            def one(g, iv, mv):
                vbase = (g >> 3) * 1024 + (g & 7) * 16
                eff = jnp.where(mv == 1, iv, jnp.int32(n_frames))

                def tg(p):
                    w = plsc.load_gather(tab_v.at[pl.ds(p * N_PAD, N_PAD)], [eff])
                    return plsc.unpack(plsc.bitcast(w, jnp.bfloat16),
                                       format=interleaved,
                                       preferred_element_type=jnp.float32)

                t0, t1 = tg(0)
                t2, m00 = tg(1)
                m01, m02 = tg(2)
                m10, m11 = tg(3)
                m12, m20 = tg(4)
                m21, m22 = tg(5)
                r0 = ray_v[pl.ds(vbase, 16)]
                r1 = ray_v[pl.ds(vbase + 128, 16)]
                r2 = ray_v[pl.ds(vbase + 256, 16)]
                r3 = ray_v[pl.ds(vbase + 384, 16)]
                r4 = ray_v[pl.ds(vbase + 512, 16)]
                r5 = ray_v[pl.ds(vbase + 640, 16)]
                r6 = ray_v[pl.ds(vbase + 768, 16)]
                r7 = ray_v[pl.ds(vbase + 896, 16)]
                out_v[pl.ds(vbase, 16)] = r0 + t0
                out_v[pl.ds(vbase + 128, 16)] = r1 + t1
                out_v[pl.ds(vbase + 256, 16)] = r2 + t2
                out_v[pl.ds(vbase + 384, 16)] = m00 * r3 + m01 * r4 + m02 * r5
                out_v[pl.ds(vbase + 512, 16)] = m10 * r3 + m11 * r4 + m12 * r5
                out_v[pl.ds(vbase + 640, 16)] = m20 * r3 + m21 * r4 + m22 * r5
                out_v[pl.ds(vbase + 768, 16)] = r6
                out_v[pl.ds(vbase + 896, 16)] = r7

            def grp(j, carry):
                for u in range(unroll):
                    g = j * unroll + u
                    iv, mv = carry
                    noff = g * 16 + 16
                    carry = (idx_v[pl.ds(noff, 16)], msk_v[pl.ds(noff, 16)])
                    one(g, iv, mv)
                return carry

            carry0 = (idx_v[pl.ds(0, 16)], msk_v[pl.ds(0, 16)])
            lax.fori_loop(0, tiles_per_chunk * 8 // unroll, grp, carry0)

            if ci == 0:
                @pl.when(wid == 0)
                def _fixup():
                    m6 = iota < 6
                    plsc.store_scatter(out_v, [iota + 6 * 128], zero16, mask=m6)
                    plsc.store_scatter(out_v, [iota + 7 * 128], zero16, mask=m6)

            wbase = wid * tiles_w * 1024 + ci * cw
            out_handles[ci] = pltpu.async_copy(
                out_v, out_hbm.at[pl.ds(wbase, cw)], out_sems[b])
            if ci + n_buf < n_chunks:
                in_handles[ci + n_buf] = fire_in(ci + n_buf)

        for h in out_handles.values():
            h.wait()

    return sc_k


def kernel(correction_dict, rays, image_indices, depth_mask):
    n_frames, _ = correction_dict.shape
    n_rays = rays.shape[0]
    n_tiles = n_rays // 128
    corr_t = jnp.zeros((8, N_PAD), jnp.float32)
    corr_t = corr_t.at[:6, :n_frames].set(correction_dict.astype(jnp.float32).T)
    table = _table_call(corr_t).reshape(-1)
    idx = image_indices.astype(jnp.int32)
    msk = depth_mask.reshape(-1).astype(jnp.int32)
    rays_t = rays.astype(jnp.float32).reshape(n_tiles, 128, 8)
    rays_t = rays_t.transpose(0, 2, 1).reshape(-1)
    sc_k = _make_sc(n_tiles, n_frames, 8, 4)
    out = sc_k(table, rays_t, idx, msk)
    out = out.reshape(n_tiles, 8, 128).transpose(0, 2, 1)
    return out.reshape(n_rays, 8)


# speedup vs baseline: 1.0566x; 1.0566x over previous
import functools

import jax
import jax.numpy as jnp
from jax import lax
from jax.experimental import pallas as pl
from jax.experimental.pallas import tpu as pltpu
from jax.experimental.pallas import tpu_sc as plsc

N_PAD = 2048
TP = 6


def _rne_bf16_bits(bits):
    bits = bits.astype(jnp.uint32)
    return (bits + 0x7FFF + ((bits >> 16) & 1)) >> 16


def _table_body(c_ref, o_ref):
    c = c_ref[...]
    rho0, rho1, rho2 = c[0:1], c[1:2], c[2:3]
    x, y, z = c[3:4], c[4:5], c[5:6]
    th2 = x * x + y * y + z * z
    small = th2 < 1e-8
    th2s = jnp.where(small, 1.0, th2)
    th = jnp.sqrt(th2s)
    s = jnp.sin(th)
    co = jnp.cos(th)
    A = jnp.where(small, 1.0 - th2 / 6.0, s / th)
    B = jnp.where(small, 0.5 - th2 / 24.0, (1.0 - co) / th2s)
    Cc = jnp.where(small, 1.0 / 6.0 - th2 / 120.0, (th - s) / (th2s * th))
    xx, yy, zz = x * x, y * y, z * z
    xy, xz, yz = x * y, x * z, y * z
    R00 = 1.0 - B * (yy + zz)
    R01 = -A * z + B * xy
    R02 = A * y + B * xz
    R10 = A * z + B * xy
    R11 = 1.0 - B * (xx + zz)
    R12 = -A * x + B * yz
    R20 = -A * y + B * xz
    R21 = A * x + B * yz
    R22 = 1.0 - B * (xx + yy)
    J00 = 1.0 - Cc * (yy + zz)
    J01 = -B * z + Cc * xy
    J02 = B * y + Cc * xz
    J10 = B * z + Cc * xy
    J11 = 1.0 - Cc * (xx + zz)
    J12 = -B * x + Cc * yz
    J20 = -B * y + Cc * xz
    J21 = B * x + Cc * yz
    J22 = 1.0 - Cc * (xx + yy)
    t0 = J00 * rho0 + J01 * rho1 + J02 * rho2
    t1 = J10 * rho0 + J11 * rho1 + J12 * rho2
    t2 = J20 * rho0 + J21 * rho1 + J22 * rho2
    rows = (t0, t1, t2, R00, R01, R02, R10, R11, R12, R20, R21, R22)
    for p in range(TP):
        lo = _rne_bf16_bits(lax.bitcast_convert_type(rows[2 * p], jnp.uint32))
        hi = _rne_bf16_bits(lax.bitcast_convert_type(rows[2 * p + 1], jnp.uint32))
        packed = lax.bitcast_convert_type(lo | (hi << 16), jnp.int32)
        o_ref[pl.ds(p * N_PAD, N_PAD)] = packed.reshape(N_PAD)


_table_call = pl.pallas_call(
    _table_body,
    out_shape=jax.ShapeDtypeStruct((TP * N_PAD,), jnp.int32),
)


@functools.lru_cache(maxsize=None)
def _make_sc(n_tiles, n_frames, n_chunks, unroll, n_buf=4):
    info = plsc.get_sparse_core_info()
    nc, ns = info.num_cores, info.num_subcores
    nw = nc * ns
    tiles_w = n_tiles // nw
    assert tiles_w * nw == n_tiles and tiles_w % n_chunks == 0
    tiles_per_chunk = tiles_w // n_chunks
    cw = tiles_per_chunk * 1024
    ciw = tiles_per_chunk * 128
    n_rays_w = tiles_w * 128
    mesh = plsc.VectorSubcoreMesh(core_axis_name="c", subcore_axis_name="s")
    interleaved = plsc.PackFormat.INTERLEAVED

    @functools.partial(
        pl.kernel,
        out_type=jax.ShapeDtypeStruct((n_tiles * 1024,), jnp.float32),
        mesh=mesh,
        compiler_params=pltpu.CompilerParams(needs_layout_passes=False),
        scratch_types=[
            pltpu.VMEM((N_PAD * TP,), jnp.int32),
            [pltpu.VMEM((ciw + 16,), jnp.int32) for _ in range(n_buf)],
            [pltpu.VMEM((ciw + 16,), jnp.int32) for _ in range(n_buf)],
            [pltpu.VMEM((cw,), jnp.float32) for _ in range(n_buf)],
            [pltpu.VMEM((cw,), jnp.float32) for _ in range(n_buf)],
            [pltpu.SemaphoreType.DMA for _ in range(n_buf)],
            [pltpu.SemaphoreType.DMA for _ in range(n_buf)],
        ],
    )
    def sc_k(table_hbm, rays_hbm, idx_hbm, msk_hbm, out_hbm,
             tab_v, idx_vs, msk_vs, ray_vs, out_vs, in_sems, out_sems):
        wid = lax.axis_index("s") * nc + lax.axis_index("c")
        iota = lax.iota(jnp.int32, 16)
        zero16 = jnp.zeros((16,), jnp.float32)

        def fire_in(ci):
            b = ci % n_buf
            rbase = wid * n_rays_w + ci * ciw
            wbase = wid * tiles_w * 1024 + ci * cw
            return (
                pltpu.async_copy(idx_hbm.at[pl.ds(rbase, ciw)],
                                 idx_vs[b].at[pl.ds(0, ciw)], in_sems[b]),
                pltpu.async_copy(msk_hbm.at[pl.ds(rbase, ciw)],
                                 msk_vs[b].at[pl.ds(0, ciw)], in_sems[b]),
                pltpu.async_copy(rays_hbm.at[pl.ds(wbase, cw)], ray_vs[b], in_sems[b]),
            )

        in_handles = {ci: fire_in(ci) for ci in range(min(n_buf, n_chunks))}
        pltpu.sync_copy(table_hbm, tab_v)

        out_handles = {}
        for ci in range(n_chunks):
            b = ci % n_buf
            idx_v, msk_v = idx_vs[b], msk_vs[b]
            ray_v, out_v = ray_vs[b], out_vs[b]
            for h in in_handles.pop(ci):
                h.wait()
            if ci - n_buf >= 0:
                out_handles.pop(ci - n_buf).wait()

            def one(g, iv, mv):
                vbase = (g >> 3) * 1024 + (g & 7) * 16
                eff = jnp.where(mv == 1, iv, jnp.int32(n_frames))

                def tg(p):
                    w = plsc.load_gather(tab_v.at[pl.ds(p * N_PAD, N_PAD)], [eff])
                    return plsc.unpack(plsc.bitcast(w, jnp.bfloat16),
                                       format=interleaved,
                                       preferred_element_type=jnp.float32)

                t0, t1 = tg(0)
                t2, m00 = tg(1)
                m01, m02 = tg(2)
                m10, m11 = tg(3)
                m12, m20 = tg(4)
                m21, m22 = tg(5)
                r0 = ray_v[pl.ds(vbase, 16)]
                r1 = ray_v[pl.ds(vbase + 128, 16)]
                r2 = ray_v[pl.ds(vbase + 256, 16)]
                r3 = ray_v[pl.ds(vbase + 384, 16)]
                r4 = ray_v[pl.ds(vbase + 512, 16)]
                r5 = ray_v[pl.ds(vbase + 640, 16)]
                r6 = ray_v[pl.ds(vbase + 768, 16)]
                r7 = ray_v[pl.ds(vbase + 896, 16)]
                out_v[pl.ds(vbase, 16)] = r0 + t0
                out_v[pl.ds(vbase + 128, 16)] = r1 + t1
                out_v[pl.ds(vbase + 256, 16)] = r2 + t2
                out_v[pl.ds(vbase + 384, 16)] = m00 * r3 + m01 * r4 + m02 * r5
                out_v[pl.ds(vbase + 512, 16)] = m10 * r3 + m11 * r4 + m12 * r5
                out_v[pl.ds(vbase + 640, 16)] = m20 * r3 + m21 * r4 + m22 * r5
                out_v[pl.ds(vbase + 768, 16)] = r6
                out_v[pl.ds(vbase + 896, 16)] = r7

            def grp(j, carry):
                for u in range(unroll):
                    g = j * unroll + u
                    iv, mv = carry
                    noff = g * 16 + 16
                    carry = (idx_v[pl.ds(noff, 16)], msk_v[pl.ds(noff, 16)])
                    one(g, iv, mv)
                return carry

            carry0 = (idx_v[pl.ds(0, 16)], msk_v[pl.ds(0, 16)])
            lax.fori_loop(0, tiles_per_chunk * 8 // unroll, grp, carry0)

            if ci == 0:
                @pl.when(wid == 0)
                def _fixup():
                    m6 = iota < 6
                    plsc.store_scatter(out_v, [iota + 6 * 128], zero16, mask=m6)
                    plsc.store_scatter(out_v, [iota + 7 * 128], zero16, mask=m6)

            wbase = wid * tiles_w * 1024 + ci * cw
            out_handles[ci] = pltpu.async_copy(
                out_v, out_hbm.at[pl.ds(wbase, cw)], out_sems[b])
            if ci + n_buf < n_chunks:
                in_handles[ci + n_buf] = fire_in(ci + n_buf)

        for h in out_handles.values():
            h.wait()

    return sc_k


def kernel(correction_dict, rays, image_indices, depth_mask):
    n_frames, _ = correction_dict.shape
    n_rays = rays.shape[0]
    n_tiles = n_rays // 128
    corr_t = jnp.zeros((8, N_PAD), jnp.float32)
    corr_t = corr_t.at[:6, :n_frames].set(correction_dict.astype(jnp.float32).T)
    table = _table_call(corr_t)
    idx = image_indices.astype(jnp.int32)
    msk = depth_mask.reshape(-1).astype(jnp.int32)
    rays_t = rays.astype(jnp.float32).reshape(n_tiles, 128, 8)
    rays_t = rays_t.transpose(0, 2, 1).reshape(-1)
    sc_k = _make_sc(n_tiles, n_frames, 8, 2)
    out = sc_k(table, rays_t, idx, msk)
    out = out.reshape(n_tiles, 8, 128).transpose(0, 2, 1)
    return out.reshape(n_rays, 8)
